# baseline (speedup 1.0000x reference)
import jax
import jax.numpy as jnp
from jax import lax
from jax.experimental import pallas as pl
from jax.experimental.pallas import tpu as pltpu
from jax.experimental.pallas import tpu_sc as plsc

T = 512
D = 1024
H = 512
E = 16
TILE = 128
NT = (T + E * (TILE - 1)) // TILE
TPAD = NT * TILE

NC, NS = 2, 16
NW = NC * NS
CHUNK = T // NW


def _route_body(x_ref, gw_ref, gb_ref, pos_ref, te_ref):
    x = x_ref[...]
    logits = jnp.dot(x, gw_ref[...], preferred_element_type=jnp.float32)
    logits = logits + gb_ref[...]
    spike = logits > 1.0
    eids = lax.broadcasted_iota(jnp.int32, (T, E), 1)
    eid = jnp.min(jnp.where(spike, eids, E), axis=1, keepdims=True)
    eid = jnp.where(eid == E, 0, eid)
    onehot = (eids == eid).astype(jnp.bfloat16)
    onehot_f = onehot.astype(jnp.float32)
    counts = jnp.sum(onehot_f, axis=0, keepdims=True)
    rr = lax.broadcasted_iota(jnp.int32, (T, T), 0)
    cc = lax.broadcasted_iota(jnp.int32, (T, T), 1)
    ltri = (cc < rr).astype(jnp.bfloat16)
    pref = jnp.dot(ltri, onehot, preferred_element_type=jnp.float32)
    rank = jnp.sum(pref * onehot_f, axis=1, keepdims=True)
    padc = jnp.ceil(counts * (1.0 / TILE)) * TILE
    er = lax.broadcasted_iota(jnp.int32, (E, E), 0)
    ec = lax.broadcasted_iota(jnp.int32, (E, E), 1)
    utri = (er < ec).astype(jnp.float32)
    aoff = jnp.dot(padc, utri, preferred_element_type=jnp.float32)
    pos = jnp.sum(onehot_f * aoff, axis=1, keepdims=True) + rank
    pos_ref[...] = pos.astype(jnp.int32)
    used = jnp.sum(padc) * (1.0 / TILE)
    atile = aoff * (1.0 / TILE)
    tn = lax.broadcasted_iota(jnp.int32, (NT, E), 0).astype(jnp.float32)
    tn = jnp.minimum(tn, used - 1.0)
    cmp = (jnp.broadcast_to(atile, (NT, E)) <= tn).astype(jnp.int32)
    te_ref[...] = jnp.sum(cmp, axis=1, keepdims=True) - 1


def _route(x_flat, gate_W, gate_b2):
    return pl.pallas_call(
        _route_body,
        out_shape=[
            jax.ShapeDtypeStruct((T, 1), jnp.int32),
            jax.ShapeDtypeStruct((NT, 1), jnp.int32),
        ],
    )(x_flat, gate_W, gate_b2)


def _dispatch_body(pos_hbm, x_hbm, xs_hbm, posv, rows, sem, sem2):
    w = lax.axis_index("s") * NC + lax.axis_index("c")
    base = w * CHUNK
    xcp = pltpu.async_copy(x_hbm.at[pl.ds(base, CHUNK)], rows, sem2)
    pltpu.sync_copy(pos_hbm.at[pl.ds(base, CHUNK)], posv)
    xcp.wait()
    pltpu.async_copy(rows, xs_hbm.at[posv], sem).wait()


def _dispatch(pos, x_flat):
    return pl.kernel(
        _dispatch_body,
        out_type=jax.ShapeDtypeStruct((TPAD, D), jnp.float32),
        mesh=plsc.VectorSubcoreMesh(core_axis_name="c", subcore_axis_name="s"),
        scratch_types=[
            pltpu.VMEM((CHUNK,), jnp.int32),
            pltpu.VMEM((CHUNK, D), jnp.float32),
            pltpu.SemaphoreType.DMA,
            pltpu.SemaphoreType.DMA,
        ],
    )(pos, x_flat)


def _ffn_inner(xs_ref, wg_ref, bg_ref, wu_ref, bu_ref, wd_ref, bd_ref,
               ys_ref):
    xt = xs_ref[...]
    g = jnp.dot(xt, wg_ref[0], preferred_element_type=jnp.float32)
    g = g + bg_ref[0]
    u = jnp.dot(xt, wu_ref[0], preferred_element_type=jnp.float32)
    u = u + bu_ref[0]
    h = jnp.where(g > 1.0, u, 0.0)
    y = jnp.dot(h, wd_ref[0], preferred_element_type=jnp.float32)
    ys_ref[...] = y + bd_ref[0]


def _ffn_outer(te_ref, xs_hbm, wg_hbm, bg_hbm, wu_hbm, bu_hbm, wd_hbm,
               bd_hbm, ys_hbm):
    deep = pl.Buffered(buffer_count=4, use_lookahead=True)

    def wmap(i):
        return (te_ref[i], 0, 0)

    pltpu.emit_pipeline(
        _ffn_inner,
        grid=(NT,),
        in_specs=[
            pl.BlockSpec((TILE, D), lambda i: (i, 0)),
            pl.BlockSpec((1, D, H), wmap, pipeline_mode=deep),
            pl.BlockSpec((1, 1, H), wmap),
            pl.BlockSpec((1, D, H), wmap, pipeline_mode=deep),
            pl.BlockSpec((1, 1, H), wmap),
            pl.BlockSpec((1, H, D), wmap, pipeline_mode=deep),
            pl.BlockSpec((1, 1, D), wmap),
        ],
        out_specs=[pl.BlockSpec((TILE, D), lambda i: (i, 0))],
    )(xs_hbm, wg_hbm, bg_hbm, wu_hbm, bu_hbm, wd_hbm, bd_hbm, ys_hbm)


def _ffn(te, xs, Wg, bg, Wu, bu, Wd, bd):
    return pl.pallas_call(
        _ffn_outer,
        in_specs=[
            pl.BlockSpec(memory_space=pltpu.MemorySpace.SMEM),
            pl.BlockSpec(memory_space=pltpu.MemorySpace.HBM),
            pl.BlockSpec(memory_space=pltpu.MemorySpace.HBM),
            pl.BlockSpec(memory_space=pltpu.MemorySpace.HBM),
            pl.BlockSpec(memory_space=pltpu.MemorySpace.HBM),
            pl.BlockSpec(memory_space=pltpu.MemorySpace.HBM),
            pl.BlockSpec(memory_space=pltpu.MemorySpace.HBM),
            pl.BlockSpec(memory_space=pltpu.MemorySpace.HBM),
        ],
        out_specs=pl.BlockSpec(memory_space=pltpu.MemorySpace.HBM),
        out_shape=jax.ShapeDtypeStruct((TPAD, D), jnp.float32),
    )(te, xs, Wg, bg.reshape(E, 1, H), Wu, bu.reshape(E, 1, H),
      Wd, bd.reshape(E, 1, D))


def _combine_body(pos_hbm, ys_hbm, out_hbm, posv, rows, sem):
    w = lax.axis_index("s") * NC + lax.axis_index("c")
    base = w * CHUNK
    pltpu.sync_copy(pos_hbm.at[pl.ds(base, CHUNK)], posv)
    pltpu.async_copy(ys_hbm.at[posv], rows, sem).wait()
    pltpu.sync_copy(rows, out_hbm.at[pl.ds(base, CHUNK)])


def _combine(pos, ys):
    return pl.kernel(
        _combine_body,
        out_type=jax.ShapeDtypeStruct((T, D), jnp.float32),
        mesh=plsc.VectorSubcoreMesh(core_axis_name="c", subcore_axis_name="s"),
        scratch_types=[
            pltpu.VMEM((CHUNK,), jnp.int32),
            pltpu.VMEM((CHUNK, D), jnp.float32),
            pltpu.SemaphoreType.DMA,
        ],
    )(pos, ys)


def kernel(x, gate_W, gate_b, Wg, bg, Wu, bu, Wd, bd):
    B, S, _ = x.shape
    x_flat = x.reshape(B * S, D)
    pos2, te2 = _route(x_flat, gate_W, gate_b.reshape(1, E))
    pos = pos2.reshape(T)
    te = te2.reshape(NT)
    xs = _dispatch(pos, x_flat)
    ys = _ffn(te, xs, Wg, bg, Wu, bu, Wd, bd)
    out = _combine(pos, ys)
    return out.reshape(B, S, D)


# speedup vs baseline: 1.0102x; 1.0102x over previous
import jax
import jax.numpy as jnp
from jax import lax
from jax.experimental import pallas as pl
from jax.experimental.pallas import tpu as pltpu
from jax.experimental.pallas import tpu_sc as plsc

T = 512
D = 1024
H = 512
E = 16
TILE = 128
NT = (T + E * (TILE - 1)) // TILE
TPAD = NT * TILE

NC, NS = 2, 16
NW = NC * NS
CHUNK = T // NW


def _route_body(x_ref, gw_ref, gb_ref, pos_ref, te_ref):
    x = x_ref[...]
    logits = jnp.dot(x, gw_ref[...], preferred_element_type=jnp.float32)
    logits = logits + gb_ref[...]
    spike = logits > 1.0
    eids = lax.broadcasted_iota(jnp.int32, (T, E), 1)
    eid = jnp.min(jnp.where(spike, eids, E), axis=1, keepdims=True)
    eid = jnp.where(eid == E, 0, eid)
    onehot = (eids == eid).astype(jnp.bfloat16)
    onehot_f = onehot.astype(jnp.float32)
    counts = jnp.sum(onehot_f, axis=0, keepdims=True)
    rr = lax.broadcasted_iota(jnp.int32, (T, T), 0)
    cc = lax.broadcasted_iota(jnp.int32, (T, T), 1)
    ltri = (cc < rr).astype(jnp.bfloat16)
    pref = jnp.dot(ltri, onehot, preferred_element_type=jnp.float32)
    rank = jnp.sum(pref * onehot_f, axis=1, keepdims=True)
    padc = jnp.ceil(counts * (1.0 / TILE)) * TILE
    er = lax.broadcasted_iota(jnp.int32, (E, E), 0)
    ec = lax.broadcasted_iota(jnp.int32, (E, E), 1)
    utri = (er < ec).astype(jnp.float32)
    aoff = jnp.dot(padc, utri, preferred_element_type=jnp.float32)
    pos = jnp.sum(onehot_f * aoff, axis=1, keepdims=True) + rank
    pos_ref[...] = pos.astype(jnp.int32)
    used = jnp.sum(padc) * (1.0 / TILE)
    atile = aoff * (1.0 / TILE)
    tn = lax.broadcasted_iota(jnp.int32, (NT, E), 0).astype(jnp.float32)
    tn = jnp.minimum(tn, used - 1.0)
    cmp = (jnp.broadcast_to(atile, (NT, E)) <= tn).astype(jnp.int32)
    te_ref[...] = jnp.sum(cmp, axis=1, keepdims=True) - 1


def _route(x_flat, gate_W, gate_b2):
    return pl.pallas_call(
        _route_body,
        out_shape=[
            jax.ShapeDtypeStruct((T, 1), jnp.int32),
            jax.ShapeDtypeStruct((NT, 1), jnp.int32),
        ],
    )(x_flat, gate_W, gate_b2)


def _dispatch_body(pos_hbm, x_hbm, xs_hbm, posv, rows, sem, sem2):
    w = lax.axis_index("s") * NC + lax.axis_index("c")
    base = w * CHUNK
    xcp = pltpu.async_copy(x_hbm.at[pl.ds(base, CHUNK)], rows, sem2)
    pltpu.sync_copy(pos_hbm.at[pl.ds(base, CHUNK)], posv)
    xcp.wait()
    pltpu.async_copy(rows, xs_hbm.at[posv], sem).wait()


def _dispatch(pos, x_flat):
    return pl.kernel(
        _dispatch_body,
        out_type=jax.ShapeDtypeStruct((TPAD, D), jnp.float32),
        mesh=plsc.VectorSubcoreMesh(core_axis_name="c", subcore_axis_name="s"),
        scratch_types=[
            pltpu.VMEM((CHUNK,), jnp.int32),
            pltpu.VMEM((CHUNK, D), jnp.float32),
            pltpu.SemaphoreType.DMA,
            pltpu.SemaphoreType.DMA,
        ],
    )(pos, x_flat)


def _ffn_inner(xs_ref, wg_ref, bg_ref, wu_ref, bu_ref, wd_ref, bd_ref,
               ys_ref):
    xt = xs_ref[...]
    g = jnp.dot(xt, wg_ref[0], preferred_element_type=jnp.float32)
    g = g + bg_ref[0]
    u = jnp.dot(xt, wu_ref[0], preferred_element_type=jnp.float32)
    u = u + bu_ref[0]
    h = jnp.where(g > 1.0, u, 0.0)
    y = jnp.dot(h, wd_ref[0], preferred_element_type=jnp.float32)
    ys_ref[...] = y + bd_ref[0]


def _ffn_outer(te_ref, xs_hbm, wg_hbm, bg_hbm, wu_hbm, bu_hbm, wd_hbm,
               bd_hbm, ys_hbm):
    deep = pl.Buffered(buffer_count=3, use_lookahead=True)

    def wmap(i):
        return (te_ref[i], 0, 0)

    pltpu.emit_pipeline(
        _ffn_inner,
        grid=(NT,),
        in_specs=[
            pl.BlockSpec((TILE, D), lambda i: (i, 0)),
            pl.BlockSpec((1, D, H), wmap, pipeline_mode=deep),
            pl.BlockSpec((1, 1, H), wmap),
            pl.BlockSpec((1, D, H), wmap, pipeline_mode=deep),
            pl.BlockSpec((1, 1, H), wmap),
            pl.BlockSpec((1, H, D), wmap, pipeline_mode=deep),
            pl.BlockSpec((1, 1, D), wmap),
        ],
        out_specs=[pl.BlockSpec((TILE, D), lambda i: (i, 0))],
    )(xs_hbm, wg_hbm, bg_hbm, wu_hbm, bu_hbm, wd_hbm, bd_hbm, ys_hbm)


def _ffn(te, xs, Wg, bg, Wu, bu, Wd, bd):
    return pl.pallas_call(
        _ffn_outer,
        in_specs=[
            pl.BlockSpec(memory_space=pltpu.MemorySpace.SMEM),
            pl.BlockSpec(memory_space=pltpu.MemorySpace.HBM),
            pl.BlockSpec(memory_space=pltpu.MemorySpace.HBM),
            pl.BlockSpec(memory_space=pltpu.MemorySpace.HBM),
            pl.BlockSpec(memory_space=pltpu.MemorySpace.HBM),
            pl.BlockSpec(memory_space=pltpu.MemorySpace.HBM),
            pl.BlockSpec(memory_space=pltpu.MemorySpace.HBM),
            pl.BlockSpec(memory_space=pltpu.MemorySpace.HBM),
        ],
        out_specs=pl.BlockSpec(memory_space=pltpu.MemorySpace.HBM),
        out_shape=jax.ShapeDtypeStruct((TPAD, D), jnp.float32),
    )(te, xs, Wg, bg.reshape(E, 1, H), Wu, bu.reshape(E, 1, H),
      Wd, bd.reshape(E, 1, D))


def _combine_body(pos_hbm, ys_hbm, out_hbm, posv, rows, sem):
    w = lax.axis_index("s") * NC + lax.axis_index("c")
    base = w * CHUNK
    pltpu.sync_copy(pos_hbm.at[pl.ds(base, CHUNK)], posv)
    pltpu.async_copy(ys_hbm.at[posv], rows, sem).wait()
    pltpu.sync_copy(rows, out_hbm.at[pl.ds(base, CHUNK)])


def _combine(pos, ys):
    return pl.kernel(
        _combine_body,
        out_type=jax.ShapeDtypeStruct((T, D), jnp.float32),
        mesh=plsc.VectorSubcoreMesh(core_axis_name="c", subcore_axis_name="s"),
        scratch_types=[
            pltpu.VMEM((CHUNK,), jnp.int32),
            pltpu.VMEM((CHUNK, D), jnp.float32),
            pltpu.SemaphoreType.DMA,
        ],
    )(pos, ys)


def kernel(x, gate_W, gate_b, Wg, bg, Wu, bu, Wd, bd):
    B, S, _ = x.shape
    x_flat = x.reshape(B * S, D)
    pos2, te2 = _route(x_flat, gate_W, gate_b.reshape(1, E))
    pos = pos2.reshape(T)
    te = te2.reshape(NT)
    xs = _dispatch(pos, x_flat)
    ys = _ffn(te, xs, Wg, bg, Wu, bu, Wd, bd)
    out = _combine(pos, ys)
    return out.reshape(B, S, D)


# speedup vs baseline: 1.0117x; 1.0015x over previous
import jax
import jax.numpy as jnp
from jax import lax
from jax.experimental import pallas as pl
from jax.experimental.pallas import tpu as pltpu
from jax.experimental.pallas import tpu_sc as plsc

T = 512
D = 1024
H = 512
E = 16
TILE = 128
NT = (T + E * (TILE - 1)) // TILE
TPAD = NT * TILE

NC, NS = 2, 16
NW = NC * NS
CHUNK = T // NW


def _route_body(x_ref, gw_ref, gb_ref, pos_ref, te_ref, nt_ref):
    x = x_ref[...]
    logits = jnp.dot(x, gw_ref[...], preferred_element_type=jnp.float32)
    logits = logits + gb_ref[...]
    spike = logits > 1.0
    eids = lax.broadcasted_iota(jnp.int32, (T, E), 1)
    eid = jnp.min(jnp.where(spike, eids, E), axis=1, keepdims=True)
    eid = jnp.where(eid == E, 0, eid)
    onehot = (eids == eid).astype(jnp.bfloat16)
    onehot_f = onehot.astype(jnp.float32)
    counts = jnp.sum(onehot_f, axis=0, keepdims=True)
    rr = lax.broadcasted_iota(jnp.int32, (T, T), 0)
    cc = lax.broadcasted_iota(jnp.int32, (T, T), 1)
    ltri = (cc < rr).astype(jnp.bfloat16)
    pref = jnp.dot(ltri, onehot, preferred_element_type=jnp.float32)
    rank = jnp.sum(pref * onehot_f, axis=1, keepdims=True)
    padc = jnp.ceil(counts * (1.0 / TILE)) * TILE
    er = lax.broadcasted_iota(jnp.int32, (E, E), 0)
    ec = lax.broadcasted_iota(jnp.int32, (E, E), 1)
    utri = (er < ec).astype(jnp.float32)
    aoff = jnp.dot(padc, utri, preferred_element_type=jnp.float32)
    pos = jnp.sum(onehot_f * aoff, axis=1, keepdims=True) + rank
    pos_ref[...] = pos.astype(jnp.int32)
    used = jnp.sum(padc) * (1.0 / TILE)
    atile = aoff * (1.0 / TILE)
    tn = lax.broadcasted_iota(jnp.int32, (NT, E), 0).astype(jnp.float32)
    tn = jnp.minimum(tn, used - 1.0)
    cmp = (jnp.broadcast_to(atile, (NT, E)) <= tn).astype(jnp.int32)
    te_ref[...] = jnp.sum(cmp, axis=1, keepdims=True) - 1
    nt_ref[...] = jnp.full((1, 1), 0, jnp.int32) + used.astype(jnp.int32)


def _route(x_flat, gate_W, gate_b2):
    return pl.pallas_call(
        _route_body,
        out_shape=[
            jax.ShapeDtypeStruct((T, 1), jnp.int32),
            jax.ShapeDtypeStruct((NT, 1), jnp.int32),
            jax.ShapeDtypeStruct((1, 1), jnp.int32),
        ],
    )(x_flat, gate_W, gate_b2)


def _dispatch_body(pos_hbm, x_hbm, xs_hbm, posv, rows, sem, sem2):
    w = lax.axis_index("s") * NC + lax.axis_index("c")
    base = w * CHUNK
    xcp = pltpu.async_copy(x_hbm.at[pl.ds(base, CHUNK)], rows, sem2)
    pltpu.sync_copy(pos_hbm.at[pl.ds(base, CHUNK)], posv)
    xcp.wait()
    pltpu.async_copy(rows, xs_hbm.at[posv], sem).wait()


def _dispatch(pos, x_flat):
    return pl.kernel(
        _dispatch_body,
        out_type=jax.ShapeDtypeStruct((TPAD, D), jnp.float32),
        mesh=plsc.VectorSubcoreMesh(core_axis_name="c", subcore_axis_name="s"),
        scratch_types=[
            pltpu.VMEM((CHUNK,), jnp.int32),
            pltpu.VMEM((CHUNK, D), jnp.float32),
            pltpu.SemaphoreType.DMA,
            pltpu.SemaphoreType.DMA,
        ],
    )(pos, x_flat)


def _ffn_inner(xs_ref, wg_ref, bg_ref, wu_ref, bu_ref, wd_ref, bd_ref,
               ys_ref):
    xt = xs_ref[...]
    g = jnp.dot(xt, wg_ref[0], preferred_element_type=jnp.float32)
    g = g + bg_ref[0]
    u = jnp.dot(xt.astype(jnp.bfloat16), wu_ref[0].astype(jnp.bfloat16),
                preferred_element_type=jnp.float32)
    u = u + bu_ref[0]
    h = jnp.where(g > 1.0, u, 0.0)
    y = jnp.dot(h.astype(jnp.bfloat16), wd_ref[0].astype(jnp.bfloat16),
                preferred_element_type=jnp.float32)
    ys_ref[...] = y + bd_ref[0]


def _ffn_outer(te_ref, nt_ref, xs_hbm, wg_hbm, bg_hbm, wu_hbm, bu_hbm, wd_hbm,
               bd_hbm, ys_hbm):
    deep = pl.Buffered(buffer_count=3, use_lookahead=True)

    def wmap(i):
        return (te_ref[i], 0, 0)

    pltpu.emit_pipeline(
        _ffn_inner,
        grid=(nt_ref[0],),
        in_specs=[
            pl.BlockSpec((TILE, D), lambda i: (i, 0)),
            pl.BlockSpec((1, D, H), wmap, pipeline_mode=deep),
            pl.BlockSpec((1, 1, H), wmap),
            pl.BlockSpec((1, D, H), wmap, pipeline_mode=deep),
            pl.BlockSpec((1, 1, H), wmap),
            pl.BlockSpec((1, H, D), wmap, pipeline_mode=deep),
            pl.BlockSpec((1, 1, D), wmap),
        ],
        out_specs=[pl.BlockSpec((TILE, D), lambda i: (i, 0))],
    )(xs_hbm, wg_hbm, bg_hbm, wu_hbm, bu_hbm, wd_hbm, bd_hbm, ys_hbm)


def _ffn(te, nt, xs, Wg, bg, Wu, bu, Wd, bd):
    return pl.pallas_call(
        _ffn_outer,
        in_specs=[
            pl.BlockSpec(memory_space=pltpu.MemorySpace.SMEM),
            pl.BlockSpec(memory_space=pltpu.MemorySpace.SMEM),
            pl.BlockSpec(memory_space=pltpu.MemorySpace.HBM),
            pl.BlockSpec(memory_space=pltpu.MemorySpace.HBM),
            pl.BlockSpec(memory_space=pltpu.MemorySpace.HBM),
            pl.BlockSpec(memory_space=pltpu.MemorySpace.HBM),
            pl.BlockSpec(memory_space=pltpu.MemorySpace.HBM),
            pl.BlockSpec(memory_space=pltpu.MemorySpace.HBM),
            pl.BlockSpec(memory_space=pltpu.MemorySpace.HBM),
        ],
        out_specs=pl.BlockSpec(memory_space=pltpu.MemorySpace.HBM),
        out_shape=jax.ShapeDtypeStruct((TPAD, D), jnp.float32),
    )(te, nt, xs, Wg, bg.reshape(E, 1, H), Wu, bu.reshape(E, 1, H),
      Wd, bd.reshape(E, 1, D))


def _combine_body(pos_hbm, ys_hbm, out_hbm, posv, rows, sem):
    w = lax.axis_index("s") * NC + lax.axis_index("c")
    base = w * CHUNK
    pltpu.sync_copy(pos_hbm.at[pl.ds(base, CHUNK)], posv)
    pltpu.async_copy(ys_hbm.at[posv], rows, sem).wait()
    pltpu.sync_copy(rows, out_hbm.at[pl.ds(base, CHUNK)])


def _combine(pos, ys):
    return pl.kernel(
        _combine_body,
        out_type=jax.ShapeDtypeStruct((T, D), jnp.float32),
        mesh=plsc.VectorSubcoreMesh(core_axis_name="c", subcore_axis_name="s"),
        scratch_types=[
            pltpu.VMEM((CHUNK,), jnp.int32),
            pltpu.VMEM((CHUNK, D), jnp.float32),
            pltpu.SemaphoreType.DMA,
        ],
    )(pos, ys)


def kernel(x, gate_W, gate_b, Wg, bg, Wu, bu, Wd, bd):
    B, S, _ = x.shape
    x_flat = x.reshape(B * S, D)
    pos2, te2, nt2 = _route(x_flat, gate_W, gate_b.reshape(1, E))
    pos = pos2.reshape(T)
    te = te2.reshape(NT)
    xs = _dispatch(pos, x_flat)
    ys = _ffn(te, nt2.reshape(1), xs, Wg, bg, Wu, bu, Wd, bd)
    out = _combine(pos, ys)
    return out.reshape(B, S, D)


# speedup vs baseline: 1.0390x; 1.0270x over previous
import jax
import jax.numpy as jnp
from jax import lax
from jax.experimental import pallas as pl
from jax.experimental.pallas import tpu as pltpu
from jax.experimental.pallas import tpu_sc as plsc

T = 512
D = 1024
H = 512
E = 16
TILE = 128
NT = (T + E * (TILE - 1)) // TILE
TPAD = NT * TILE

NC, NS = 2, 16
NW = NC * NS
CHUNK = T // NW


def _route_body(x_ref, gw_ref, gb_ref, pos_ref, te_ref, nt_ref):
    x = x_ref[...]
    logits = jnp.dot(x, gw_ref[...], preferred_element_type=jnp.float32)
    logits = logits + gb_ref[...]
    spike = logits > 1.0
    eids = lax.broadcasted_iota(jnp.int32, (T, E), 1)
    eid = jnp.min(jnp.where(spike, eids, E), axis=1, keepdims=True)
    eid = jnp.where(eid == E, 0, eid)
    onehot = (eids == eid).astype(jnp.bfloat16)
    onehot_f = onehot.astype(jnp.float32)
    counts = jnp.sum(onehot_f, axis=0, keepdims=True)
    rr = lax.broadcasted_iota(jnp.int32, (T, T), 0)
    cc = lax.broadcasted_iota(jnp.int32, (T, T), 1)
    ltri = (cc < rr).astype(jnp.bfloat16)
    pref = jnp.dot(ltri, onehot, preferred_element_type=jnp.float32)
    rank = jnp.sum(pref * onehot_f, axis=1, keepdims=True)
    padc = jnp.ceil(counts * (1.0 / TILE)) * TILE
    er = lax.broadcasted_iota(jnp.int32, (E, E), 0)
    ec = lax.broadcasted_iota(jnp.int32, (E, E), 1)
    utri = (er < ec).astype(jnp.float32)
    aoff = jnp.dot(padc, utri, preferred_element_type=jnp.float32)
    pos = jnp.sum(onehot_f * aoff, axis=1, keepdims=True) + rank
    pos_ref[...] = pos.astype(jnp.int32)
    used = jnp.sum(padc) * (1.0 / TILE)
    atile = aoff * (1.0 / TILE)
    tn = lax.broadcasted_iota(jnp.int32, (NT, E), 0).astype(jnp.float32)
    tn = jnp.minimum(tn, used - 1.0)
    cmp = (jnp.broadcast_to(atile, (NT, E)) <= tn).astype(jnp.int32)
    te_ref[...] = jnp.sum(cmp, axis=1, keepdims=True) - 1
    nt_ref[...] = jnp.full((1, 1), 0, jnp.int32) + used.astype(jnp.int32)


def _route(x_flat, gate_W, gate_b2):
    return pl.pallas_call(
        _route_body,
        out_shape=[
            jax.ShapeDtypeStruct((T, 1), jnp.int32),
            jax.ShapeDtypeStruct((NT, 1), jnp.int32),
            jax.ShapeDtypeStruct((1, 1), jnp.int32),
        ],
    )(x_flat, gate_W, gate_b2)


def _dispatch_body(pos_hbm, x_hbm, xs_hbm, posv, rows, sem, sem2):
    w = lax.axis_index("s") * NC + lax.axis_index("c")
    base = w * CHUNK
    xcp = pltpu.async_copy(x_hbm.at[pl.ds(base, CHUNK)], rows, sem2)
    pltpu.sync_copy(pos_hbm.at[pl.ds(base, CHUNK)], posv)
    xcp.wait()
    pltpu.async_copy(rows, xs_hbm.at[posv], sem).wait()


def _dispatch(pos, x_flat):
    return pl.kernel(
        _dispatch_body,
        out_type=jax.ShapeDtypeStruct((TPAD, D), jnp.float32),
        mesh=plsc.VectorSubcoreMesh(core_axis_name="c", subcore_axis_name="s"),
        scratch_types=[
            pltpu.VMEM((CHUNK,), jnp.int32),
            pltpu.VMEM((CHUNK, D), jnp.float32),
            pltpu.SemaphoreType.DMA,
            pltpu.SemaphoreType.DMA,
        ],
    )(pos, x_flat)


def _ffn_inner(xs_ref, wg_ref, bg_ref, wu_ref, bu_ref, wd_ref, bd_ref,
               ys_ref):
    xt = xs_ref[...]
    g = jnp.dot(xt, wg_ref[0], preferred_element_type=jnp.float32)
    g = g + bg_ref[0]
    u = jnp.dot(xt.astype(jnp.bfloat16), wu_ref[0].astype(jnp.bfloat16),
                preferred_element_type=jnp.float32)
    u = u + bu_ref[0]
    h = jnp.where(g > 1.0, u, 0.0)
    y = jnp.dot(h.astype(jnp.bfloat16), wd_ref[0].astype(jnp.bfloat16),
                preferred_element_type=jnp.float32)
    ys_ref[...] = y + bd_ref[0]


def _ffn_outer(te_ref, nt_ref, xs_hbm, wg_hbm, bg_hbm, wu_hbm, bu_hbm, wd_hbm,
               bd_hbm, ys_hbm):
    deep = pl.Buffered(buffer_count=3, use_lookahead=True)

    def wmap(i):
        return (te_ref[i], 0, 0)

    pltpu.emit_pipeline(
        _ffn_inner,
        grid=(nt_ref[0],),
        in_specs=[
            pl.BlockSpec((TILE, D), lambda i: (i, 0),
                         pipeline_mode=pl.Buffered(buffer_count=3)),
            pl.BlockSpec((1, D, H), wmap, pipeline_mode=deep),
            pl.BlockSpec((1, 1, H), wmap),
            pl.BlockSpec((1, D, H), wmap, pipeline_mode=deep),
            pl.BlockSpec((1, 1, H), wmap),
            pl.BlockSpec((1, H, D), wmap, pipeline_mode=deep),
            pl.BlockSpec((1, 1, D), wmap),
        ],
        out_specs=[pl.BlockSpec((TILE, D), lambda i: (i, 0))],
    )(xs_hbm, wg_hbm, bg_hbm, wu_hbm, bu_hbm, wd_hbm, bd_hbm, ys_hbm)


def _ffn(te, nt, xs, Wg, bg, Wu, bu, Wd, bd):
    return pl.pallas_call(
        _ffn_outer,
        in_specs=[
            pl.BlockSpec(memory_space=pltpu.MemorySpace.SMEM),
            pl.BlockSpec(memory_space=pltpu.MemorySpace.SMEM),
            pl.BlockSpec(memory_space=pltpu.MemorySpace.HBM),
            pl.BlockSpec(memory_space=pltpu.MemorySpace.HBM),
            pl.BlockSpec(memory_space=pltpu.MemorySpace.HBM),
            pl.BlockSpec(memory_space=pltpu.MemorySpace.HBM),
            pl.BlockSpec(memory_space=pltpu.MemorySpace.HBM),
            pl.BlockSpec(memory_space=pltpu.MemorySpace.HBM),
            pl.BlockSpec(memory_space=pltpu.MemorySpace.HBM),
        ],
        out_specs=pl.BlockSpec(memory_space=pltpu.MemorySpace.HBM),
        out_shape=jax.ShapeDtypeStruct((TPAD, D), jnp.float32),
    )(te, nt, xs, Wg, bg.reshape(E, 1, H), Wu, bu.reshape(E, 1, H),
      Wd, bd.reshape(E, 1, D))


def _combine_body(pos_hbm, ys_hbm, out_hbm, posv, rows, sem):
    w = lax.axis_index("s") * NC + lax.axis_index("c")
    base = w * CHUNK
    pltpu.sync_copy(pos_hbm.at[pl.ds(base, CHUNK)], posv)
    pltpu.async_copy(ys_hbm.at[posv], rows, sem).wait()
    pltpu.sync_copy(rows, out_hbm.at[pl.ds(base, CHUNK)])


def _combine(pos, ys):
    return pl.kernel(
        _combine_body,
        out_type=jax.ShapeDtypeStruct((T, D), jnp.float32),
        mesh=plsc.VectorSubcoreMesh(core_axis_name="c", subcore_axis_name="s"),
        scratch_types=[
            pltpu.VMEM((CHUNK,), jnp.int32),
            pltpu.VMEM((CHUNK, D), jnp.float32),
            pltpu.SemaphoreType.DMA,
        ],
    )(pos, ys)


def kernel(x, gate_W, gate_b, Wg, bg, Wu, bu, Wd, bd):
    B, S, _ = x.shape
    x_flat = x.reshape(B * S, D)
    pos2, te2, nt2 = _route(x_flat, gate_W, gate_b.reshape(1, E))
    pos = pos2.reshape(T)
    te = te2.reshape(NT)
    xs = _dispatch(pos, x_flat)
    ys = _ffn(te, nt2.reshape(1), xs, Wg, bg, Wu, bu, Wd, bd)
    out = _combine(pos, ys)
    return out.reshape(B, S, D)
